# baseline (speedup 1.0000x reference)
import functools

import jax
import jax.numpy as jnp
from jax import lax
from jax.experimental import pallas as pl
from jax.experimental.pallas import tpu as pltpu
from jax.experimental.pallas import tpu_sc as plsc

N = 3200000
NC = 2
NS = 16
NW = NC * NS
LANES = 16

MS = 1638400
CHUNK = MS // NW
NSUB = 5
SUB = CHUNK // NSUB
UNROLL = 5
SITERS = SUB // (UNROLL * LANES)

MT = N - MS
ROWS = N // 1024
TOFF_ROWS = MS // 1024
TBLK = 25
TGRID = (ROWS - TOFF_ROWS) // TBLK
TOFF = TOFF_ROWS // TBLK

_mesh = plsc.VectorSubcoreMesh(core_axis_name="c", subcore_axis_name="s")


@functools.partial(
    pl.kernel,
    out_type=jax.ShapeDtypeStruct((NW, LANES), jnp.float32),
    mesh=_mesh,
    scratch_types=[
        [pltpu.VMEM((SUB,), jnp.float32) for _ in range(NSUB)],
        pltpu.VMEM((LANES,), jnp.float32),
        [pltpu.SemaphoreType.DMA for _ in range(NSUB)],
    ],
)
def _partial_sums(x_hbm, out_hbm, bufs, part, sems):
    wid = lax.axis_index("s") * NC + lax.axis_index("c")
    base = wid * CHUNK

    copies = [
        pltpu.make_async_copy(
            x_hbm.at[pl.ds(base + k * SUB, SUB)], bufs[k], sems[k]
        )
        for k in range(NSUB)
    ]
    for k in range(NSUB):
        copies[k].start()

    total = jnp.zeros((LANES,), jnp.float32)
    for k in range(NSUB):
        copies[k].wait()

        def body(i, accs, buf=bufs[k]):
            off = i * (UNROLL * LANES)
            return tuple(
                accs[j] + buf[pl.ds(off + j * LANES, LANES)]
                for j in range(UNROLL)
            )

        zero = jnp.zeros((LANES,), jnp.float32)
        accs = lax.fori_loop(0, SITERS, body, (zero,) * UNROLL)
        for j in range(UNROLL):
            total = total + accs[j]

    part[...] = total
    pltpu.sync_copy(part, out_hbm.at[wid])


def _tc_reduce_body(x_ref, out_ref):
    @pl.when(pl.program_id(0) == 0)
    def _():
        out_ref[...] = jnp.zeros_like(out_ref)

    out_ref[...] += jnp.sum(x_ref[...], axis=0)


_tc_reduce = pl.pallas_call(
    _tc_reduce_body,
    grid=(TGRID,),
    in_specs=[pl.BlockSpec((TBLK, 8, 128), lambda i: (TOFF + i, 0, 0))],
    out_specs=pl.BlockSpec((8, 128), lambda i: (0, 0)),
    out_shape=jax.ShapeDtypeStruct((8, 128), jnp.float32),
)


def _join_body(parts_ref, tcp_ref, out_ref):
    out_ref[...] = (jnp.sum(parts_ref[...]) + jnp.sum(tcp_ref[...])).reshape(
        1, 1
    )


_join = pl.pallas_call(
    _join_body,
    out_shape=jax.ShapeDtypeStruct((1, 1), jnp.float32),
)


def kernel(atomic_energy):
    x = atomic_energy.reshape(-1)
    parts = _partial_sums(x)
    tc_part = _tc_reduce(atomic_energy.reshape(ROWS, 8, 128))
    return _join(parts, tc_part)


# speedup vs baseline: 1.8715x; 1.8715x over previous
import functools

import jax
import jax.numpy as jnp
from jax import lax
from jax.experimental import pallas as pl
from jax.experimental.pallas import tpu as pltpu
from jax.experimental.pallas import tpu_sc as plsc

N = 3200000
NC = 2
NS = 16
NW = NC * NS
LANES = 16

MS = 1024000
CHUNK = MS // NW
NSUB = 5
SUB = CHUNK // NSUB
UNROLL = 5
SITERS = SUB // (UNROLL * LANES)

MT = N - MS
TROWS = MT // 1024
TBLK = 425
TGRID = TROWS // TBLK

_mesh = plsc.VectorSubcoreMesh(core_axis_name="c", subcore_axis_name="s")


@functools.partial(
    pl.kernel,
    out_type=jax.ShapeDtypeStruct((NW, LANES), jnp.float32),
    mesh=_mesh,
    scratch_types=[
        [pltpu.VMEM((SUB,), jnp.float32) for _ in range(NSUB)],
        pltpu.VMEM((LANES,), jnp.float32),
        [pltpu.SemaphoreType.DMA for _ in range(NSUB)],
    ],
)
def _partial_sums(x_hbm, out_hbm, bufs, part, sems):
    wid = lax.axis_index("s") * NC + lax.axis_index("c")
    base = MT + wid * CHUNK

    copies = [
        pltpu.make_async_copy(
            x_hbm.at[pl.ds(base + k * SUB, SUB)], bufs[k], sems[k]
        )
        for k in range(NSUB)
    ]
    for k in range(NSUB):
        copies[k].start()

    total = jnp.zeros((LANES,), jnp.float32)
    for k in range(NSUB):
        copies[k].wait()

        def body(i, accs, buf=bufs[k]):
            off = i * (UNROLL * LANES)
            return tuple(
                accs[j] + buf[pl.ds(off + j * LANES, LANES)]
                for j in range(UNROLL)
            )

        zero = jnp.zeros((LANES,), jnp.float32)
        accs = lax.fori_loop(0, SITERS, body, (zero,) * UNROLL)
        for j in range(UNROLL):
            total = total + accs[j]

    part[...] = total
    pltpu.sync_copy(part, out_hbm.at[wid])


def _tc_reduce_body(x_ref, out_ref):
    @pl.when(pl.program_id(0) == 0)
    def _():
        out_ref[...] = jnp.zeros_like(out_ref)

    out_ref[...] += jnp.sum(x_ref[...], axis=0)


_tc_reduce = pl.pallas_call(
    _tc_reduce_body,
    grid=(TGRID,),
    in_specs=[pl.BlockSpec((TBLK, 8, 128), lambda i: (i, 0, 0))],
    out_specs=pl.BlockSpec((8, 128), lambda i: (0, 0)),
    out_shape=jax.ShapeDtypeStruct((8, 128), jnp.float32),
)


def _join_body(parts_ref, tcp_ref, out_ref):
    out_ref[...] = (jnp.sum(parts_ref[...]) + jnp.sum(tcp_ref[...])).reshape(
        1, 1
    )


_join = pl.pallas_call(
    _join_body,
    out_shape=jax.ShapeDtypeStruct((1, 1), jnp.float32),
)


def kernel(atomic_energy):
    x = atomic_energy.reshape(-1)
    parts = _partial_sums(x)
    tc_part = _tc_reduce(x.reshape(N // 1024, 8, 128))
    return _join(parts, tc_part)


# speedup vs baseline: 1.8723x; 1.0004x over previous
import functools

import jax
import jax.numpy as jnp
from jax import lax
from jax.experimental import pallas as pl
from jax.experimental.pallas import tpu as pltpu
from jax.experimental.pallas import tpu_sc as plsc

N = 3200000
NC = 2
NS = 16
NW = NC * NS
LANES = 16

MS = 1024000
CHUNK = MS // NW
NSUB = 5
SUB = CHUNK // NSUB
UNROLL = 5
SITERS = SUB // (UNROLL * LANES)

MT = N - MS
TROWS = MT // 128
TGRID = 5
TBLK = TROWS // TGRID

_mesh = plsc.VectorSubcoreMesh(core_axis_name="c", subcore_axis_name="s")


@functools.partial(
    pl.kernel,
    out_type=jax.ShapeDtypeStruct((NW, LANES), jnp.float32),
    mesh=_mesh,
    scratch_types=[
        [pltpu.VMEM((SUB,), jnp.float32) for _ in range(NSUB)],
        pltpu.VMEM((LANES,), jnp.float32),
        [pltpu.SemaphoreType.DMA for _ in range(NSUB)],
    ],
)
def _partial_sums(x_hbm, out_hbm, bufs, part, sems):
    wid = lax.axis_index("s") * NC + lax.axis_index("c")
    base = MT + wid * CHUNK

    copies = [
        pltpu.make_async_copy(
            x_hbm.at[pl.ds(base + k * SUB, SUB)], bufs[k], sems[k]
        )
        for k in range(NSUB)
    ]
    for k in range(NSUB):
        copies[k].start()

    total = jnp.zeros((LANES,), jnp.float32)
    for k in range(NSUB):
        copies[k].wait()

        def body(i, accs, buf=bufs[k]):
            off = i * (UNROLL * LANES)
            return tuple(
                accs[j] + buf[pl.ds(off + j * LANES, LANES)]
                for j in range(UNROLL)
            )

        zero = jnp.zeros((LANES,), jnp.float32)
        accs = lax.fori_loop(0, SITERS, body, (zero,) * UNROLL)
        for j in range(UNROLL):
            total = total + accs[j]

    part[...] = total
    pltpu.sync_copy(part, out_hbm.at[wid])


def _tc_reduce_body(x_ref, out_ref):
    @pl.when(pl.program_id(0) == 0)
    def _():
        out_ref[...] = jnp.zeros_like(out_ref)

    ones = jnp.ones((8, TBLK), jnp.float32)
    out_ref[...] += jax.lax.dot(
        ones, x_ref[...], precision=jax.lax.Precision.HIGHEST
    )


_tc_reduce = pl.pallas_call(
    _tc_reduce_body,
    grid=(TGRID,),
    in_specs=[pl.BlockSpec((TBLK, 128), lambda i: (i, 0))],
    out_specs=pl.BlockSpec((8, 128), lambda i: (0, 0)),
    out_shape=jax.ShapeDtypeStruct((8, 128), jnp.float32),
)


def _join_body(parts_ref, tcp_ref, out_ref):
    out_ref[...] = (
        jnp.sum(parts_ref[...]) + jnp.sum(tcp_ref[0:1, :])
    ).reshape(1, 1)


_join = pl.pallas_call(
    _join_body,
    out_shape=jax.ShapeDtypeStruct((1, 1), jnp.float32),
)


def kernel(atomic_energy):
    x = atomic_energy.reshape(-1)
    parts = _partial_sums(x)
    tc_part = _tc_reduce(x.reshape(N // 128, 128))
    return _join(parts, tc_part)
